# Initial kernel scaffold; baseline (speedup 1.0000x reference)
import jax
import jax.numpy as jnp
from jax.experimental import pallas as pl


def kernel(emb, W1, b1, W2, b2, edge_index):
    raise NotImplementedError("write your pallas kernel here")


# speedup vs baseline: 9.5930x; 9.5930x over previous
import functools

import jax
import jax.numpy as jnp
from jax import lax
from jax.experimental import pallas as pl
from jax.experimental.pallas import tpu as pltpu
from jax.experimental.pallas import tpu_sc as plsc

NC = 2
NS = 16
CHUNK = 128
BLK = 512


def _make_deg_kernel(np_rows, cpt):
    rpt = np_rows // NS

    def body(dstc, zrows, ones_hbm, dego, didx, ones_v, acc, gsem):
        del gsem
        cid = lax.axis_index("c")
        sid = lax.axis_index("s")
        wid = cid * NS + sid
        pltpu.sync_copy(zrows, acc.at[pl.ds(sid * rpt, rpt)])
        pltpu.sync_copy(ones_hbm, ones_v)
        pltpu.sync_copy(dstc.at[pl.ds(wid * cpt, cpt)], didx)
        plsc.subcore_barrier()

        def step(j, carry):
            pltpu.sync_copy(ones_v, acc.at[didx.at[j]], add=True)
            return carry

        lax.fori_loop(0, cpt, step, 0)
        plsc.subcore_barrier()
        pltpu.sync_copy(acc.at[pl.ds(sid * rpt, rpt)],
                        dego.at[cid, pl.ds(sid * rpt, rpt)])

    return pl.kernel(
        body,
        out_type=jax.ShapeDtypeStruct((NC, np_rows, 128), jnp.float32),
        mesh=plsc.VectorSubcoreMesh(core_axis_name="c", subcore_axis_name="s",
                                    num_cores=NC, num_subcores=NS),
        scratch_types=[
            pltpu.VMEM((cpt, CHUNK), jnp.int32),
            pltpu.VMEM((CHUNK, 128), jnp.float32),
            pltpu.VMEM_SHARED((np_rows, 128), jnp.float32),
            pltpu.SemaphoreType.DMA,
        ],
    )


GRP = 8


def _make_edge_pass_kernel(np_rows, cpt):
    rpt = np_rows // NS
    ngrp = cpt // GRP

    def body(y, srcc, dstc, zrows, sout, sidx, didx, rows, acc, gsem, isem):
        cid = lax.axis_index("c")
        sid = lax.axis_index("s")
        wid = cid * NS + sid
        base = wid * cpt
        pltpu.sync_copy(zrows, acc.at[pl.ds(sid * rpt, rpt)])
        pltpu.sync_copy(srcc.at[pl.ds(base, GRP)], sidx.at[0])
        pltpu.sync_copy(dstc.at[pl.ds(base, GRP)], didx.at[0])
        plsc.subcore_barrier()

        pltpu.async_copy(y.at[sidx.at[0, 0]], rows.at[0], gsem)

        def group(g, carry):
            gb = lax.rem(g, 2)
            ngb = lax.rem(g + 1, 2)

            @pl.when(g + 1 < ngrp)
            def _():
                off = base + (g + 1) * GRP
                pltpu.async_copy(srcc.at[pl.ds(off, GRP)], sidx.at[ngb],
                                 isem)
                pltpu.async_copy(dstc.at[pl.ds(off, GRP)], didx.at[ngb],
                                 isem)

            for k in range(GRP):
                b = k % 2
                pltpu.make_async_copy(
                    y.at[sidx.at[gb, k]], rows.at[b], gsem).wait()
                if k + 1 < GRP:
                    pltpu.async_copy(
                        y.at[sidx.at[gb, k + 1]], rows.at[1 - b], gsem)
                else:
                    @pl.when(g + 1 < ngrp)
                    def _():
                        off = base + (g + 1) * GRP
                        pltpu.make_async_copy(
                            srcc.at[pl.ds(off, GRP)], sidx.at[ngb],
                            isem).wait()
                        pltpu.make_async_copy(
                            dstc.at[pl.ds(off, GRP)], didx.at[ngb],
                            isem).wait()
                        pltpu.async_copy(
                            y.at[sidx.at[ngb, 0]], rows.at[1 - b], gsem)
                pltpu.sync_copy(rows.at[b], acc.at[didx.at[gb, k]],
                                add=True)
            return carry

        lax.fori_loop(0, ngrp, group, 0)
        plsc.subcore_barrier()
        pltpu.sync_copy(acc.at[pl.ds(sid * rpt, rpt)],
                        sout.at[cid, pl.ds(sid * rpt, rpt)])

    return pl.kernel(
        body,
        out_type=jax.ShapeDtypeStruct((NC, np_rows, 128), jnp.float32),
        mesh=plsc.VectorSubcoreMesh(core_axis_name="c", subcore_axis_name="s",
                                    num_cores=NC, num_subcores=NS),
        scratch_types=[
            pltpu.VMEM((2, GRP, CHUNK), jnp.int32),
            pltpu.VMEM((2, GRP, CHUNK), jnp.int32),
            pltpu.VMEM((2, CHUNK, 128), jnp.float32),
            pltpu.VMEM_SHARED((np_rows, 128), jnp.float32),
            pltpu.SemaphoreType.DMA,
            pltpu.SemaphoreType.DMA,
        ],
    )


def _dinv(d0, d1):
    deg = d0[0][:, 0:1] + d1[0][:, 0:1] + 1.0
    return lax.rsqrt(deg)


def _prep1_body(x, w, d0, d1, y, xw):
    dinv = _dinv(d0, d1)
    m = jnp.dot(x[...], w[...], preferred_element_type=jnp.float32)
    xw[...] = m
    y[...] = m * dinv


def _mid_body(s0, s1, xw1, d0, d1, b1, w2, y2, xw2):
    dinv = _dinv(d0, d1)
    h = jnp.maximum(
        dinv * (s0[0] + s1[0]) + (dinv * dinv) * xw1[...] + b1[...], 0.0)
    m = jnp.dot(h, w2[...], preferred_element_type=jnp.float32)
    xw2[...] = m
    y2[...] = m * dinv


def _post2_body(s0, s1, xw2, d0, d1, b2, out):
    dinv = _dinv(d0, d1)
    out[...] = jnp.maximum(
        dinv * (s0[0] + s1[0]) + (dinv * dinv) * xw2[...] + b2[...], 0.0)


def _row_spec(i):
    return (i, 0)


def _make_tc_kernels(np_rows):
    grid = (np_rows // BLK,)
    blk = lambda: pl.BlockSpec((BLK, 128), _row_spec)
    full = lambda: pl.BlockSpec((128, 128), lambda i: (0, 0))
    bias = lambda: pl.BlockSpec((1, 128), lambda i: (0, 0))
    dspec = lambda c: pl.BlockSpec((1, BLK, 128), lambda i, c=c: (c, i, 0))
    sspec = lambda c: pl.BlockSpec((1, BLK, 128), lambda i, c=c: (c, i, 0))
    two_out = [jax.ShapeDtypeStruct((np_rows, 128), jnp.float32)] * 2

    prep1 = pl.pallas_call(
        _prep1_body, grid=grid,
        in_specs=[blk(), full(), dspec(0), dspec(1)],
        out_specs=[blk(), blk()], out_shape=two_out)
    mid = pl.pallas_call(
        _mid_body, grid=grid,
        in_specs=[sspec(0), sspec(1), blk(), dspec(0), dspec(1), bias(),
                  full()],
        out_specs=[blk(), blk()], out_shape=two_out)
    post2 = pl.pallas_call(
        _post2_body, grid=grid,
        in_specs=[sspec(0), sspec(1), blk(), dspec(0), dspec(1), bias()],
        out_specs=blk(),
        out_shape=jax.ShapeDtypeStruct((np_rows, 128), jnp.float32))
    return prep1, mid, post2


@jax.jit
def kernel(emb, W1, b1, W2, b2, edge_index):
    n, d = emb.shape
    e = edge_index.shape[1]
    assert d == 128
    np_rows = -(-(n + 1) // BLK) * BLK
    per_tile = -(-e // (NC * NS * CHUNK * GRP)) * GRP
    nchunks = NC * NS * per_tile
    e_pad = nchunks * CHUNK

    pad = jnp.full((e_pad - e,), n, dtype=edge_index.dtype)
    srcc = jnp.concatenate([edge_index[0], pad]).reshape(nchunks, CHUNK)
    dstc = jnp.concatenate([edge_index[1], pad]).reshape(nchunks, CHUNK)
    emb_pad = jnp.zeros((np_rows, d), emb.dtype).at[:n].set(emb)

    rpt = np_rows // NS
    ones128 = jnp.ones((CHUNK, 128), jnp.float32)
    zrows128 = jnp.zeros((rpt, 128), jnp.float32)
    b1r = b1.reshape(1, d)
    b2r = b2.reshape(1, d)

    deg_k = _make_deg_kernel(np_rows, per_tile)
    edge_k = _make_edge_pass_kernel(np_rows, per_tile)
    prep1, mid, post2 = _make_tc_kernels(np_rows)

    deg16 = deg_k(dstc, zrows128, ones128)
    y1, xw1 = prep1(emb_pad, W1, deg16, deg16)
    s1 = edge_k(y1, srcc, dstc, zrows128)
    y2, xw2 = mid(s1, s1, xw1, deg16, deg16, b1r, W2)
    s2 = edge_k(y2, srcc, dstc, zrows128)
    out = post2(s2, s2, xw2, deg16, deg16, b2r)
    type_num = n - 1000
    return (out[:type_num], out[type_num:n])
